# baseline (speedup 1.0000x reference)
import functools

import jax
import jax.numpy as jnp
from jax import lax
from jax.experimental import pallas as pl
from jax.experimental.pallas import tpu as pltpu
from jax.experimental.pallas import tpu_sc as plsc

B, S, D = 4, 4096, 1024
L = 16

_info = plsc.get_sparse_core_info()
NC, NS = _info.num_cores, _info.num_subcores
NW = NC * NS
S_PER_W = S // NW
CHUNK = 16
N_CHUNKS = S_PER_W // CHUNK
CW = CHUNK * D

_mesh = plsc.VectorSubcoreMesh(core_axis_name="c", subcore_axis_name="s")


@functools.partial(
    pl.kernel,
    mesh=_mesh,
    out_type=jax.ShapeDtypeStruct((B * S * D,), jnp.float32),
    scratch_types=[
        pltpu.VMEM((CW,), jnp.float32),
        pltpu.VMEM((CW,), jnp.float32),
    ],
)
def _sc_add(x_hbm, pos_hbm, out_hbm, pos_v, x_v):
    wid = lax.axis_index("s") * NC + lax.axis_index("c")
    for c in range(N_CHUNKS):
        s_off = (wid * S_PER_W + c * CHUNK) * D
        pltpu.sync_copy(pos_hbm.at[pl.ds(s_off, CW)], pos_v)
        for b in range(B):
            x_off = b * (S * D) + s_off
            pltpu.sync_copy(x_hbm.at[pl.ds(x_off, CW)], x_v)

            def body(i, carry):
                o = i * L
                plsc.addupdate(x_v.at[pl.ds(o, L)], pos_v[pl.ds(o, L)])
                return carry

            lax.fori_loop(0, CW // L, body, 0)
            pltpu.sync_copy(x_v, out_hbm.at[pl.ds(x_off, CW)])


def kernel(x, pos_table):
    out = _sc_add(x.reshape(-1), pos_table.reshape(-1))
    return out.reshape(x.shape)


# speedup vs baseline: 1.3346x; 1.3346x over previous
import functools

import jax
import jax.numpy as jnp
from jax import lax
from jax.experimental import pallas as pl
from jax.experimental.pallas import tpu as pltpu
from jax.experimental.pallas import tpu_sc as plsc

B, S, D = 4, 4096, 1024
L = 16

_info = plsc.get_sparse_core_info()
NC, NS = _info.num_cores, _info.num_subcores
NW = NC * NS
S_PER_W = S // NW
CHUNK = 16
N_CHUNKS = S_PER_W // CHUNK
CW = CHUNK * D

_mesh = plsc.VectorSubcoreMesh(core_axis_name="c", subcore_axis_name="s")


@functools.partial(
    pl.kernel,
    mesh=_mesh,
    out_type=jax.ShapeDtypeStruct((B * S * D,), jnp.float32),
    scratch_types=[
        pltpu.VMEM((CW,), jnp.float32),
        pltpu.VMEM((CW,), jnp.float32),
    ],
)
def _sc_add(x_hbm, pos_hbm, out_hbm, pos_v, x_v):
    wid = lax.axis_index("s") * NC + lax.axis_index("c")
    for c in range(N_CHUNKS):
        s_off = (wid * S_PER_W + c * CHUNK) * D
        pltpu.sync_copy(pos_hbm.at[pl.ds(s_off, CW)], pos_v)
        for b in range(B):
            x_off = b * (S * D) + s_off
            pltpu.sync_copy(x_hbm.at[pl.ds(x_off, CW)], x_v)

            @plsc.parallel_loop(0, CW, step=L, unroll=8)
            def _add(o):
                plsc.addupdate(x_v.at[pl.ds(o, L)], pos_v[pl.ds(o, L)])

            pltpu.sync_copy(x_v, out_hbm.at[pl.ds(x_off, CW)])


def kernel(x, pos_table):
    out = _sc_add(x.reshape(-1), pos_table.reshape(-1))
    return out.reshape(x.shape)


# speedup vs baseline: 4.4727x; 3.3513x over previous
import functools

import jax
import jax.numpy as jnp
from jax import lax
from jax.experimental import pallas as pl
from jax.experimental.pallas import tpu as pltpu
from jax.experimental.pallas import tpu_sc as plsc

B, S, D = 4, 4096, 1024
L = 16
PPR = D // L

_info = plsc.get_sparse_core_info()
NC, NS = _info.num_cores, _info.num_subcores
NW = NC * NS
S_PER_W = S // NW
CHUNK = 16
N_CHUNKS = S_PER_W // CHUNK
NU = N_CHUNKS * B
RING = 3

_mesh = plsc.VectorSubcoreMesh(core_axis_name="c", subcore_axis_name="s")


@functools.partial(
    pl.kernel,
    mesh=_mesh,
    out_type=jax.ShapeDtypeStruct((B * S, D), jnp.float32),
    scratch_types=(
        [pltpu.VMEM((CHUNK, D), jnp.float32) for _ in range(2)]
        + [pltpu.VMEM((CHUNK, D), jnp.float32) for _ in range(RING)]
        + [pltpu.SemaphoreType.DMA for _ in range(2 + 2 * RING)]
    ),
)
def _sc_add(x_hbm, pos_hbm, out_hbm, p0, p1, x0, x1, x2,
            ps0, ps1, is0, is1, is2, os0, os1, os2):
    pos_bufs, pos_sems = [p0, p1], [ps0, ps1]
    x_bufs, in_sems, out_sems = [x0, x1, x2], [is0, is1, is2], [os0, os1, os2]

    wid = lax.axis_index("s") * NC + lax.axis_index("c")
    s_base = wid * S_PER_W

    def start_pos(c):
        i = c % 2
        return pltpu.async_copy(
            pos_hbm.at[pl.ds(s_base + c * CHUNK, CHUNK)], pos_bufs[i],
            pos_sems[i])

    def start_in(u, slot):
        c, b = divmod(u, B)
        row = b * S + s_base + c * CHUNK
        return pltpu.async_copy(
            x_hbm.at[pl.ds(row, CHUNK)], x_bufs[slot], in_sems[slot])

    def start_out(u, slot):
        c, b = divmod(u, B)
        row = b * S + s_base + c * CHUNK
        return pltpu.async_copy(
            x_bufs[slot], out_hbm.at[pl.ds(row, CHUNK)], out_sems[slot])

    pos_h = [start_pos(0), None]
    in_h = [start_in(0, 0), None, None]
    out_h = [None] * RING

    for u in range(NU):
        c, b = divmod(u, B)
        slot = u % RING
        if b == 0:
            pos_h[c % 2].wait()
            if c + 1 < N_CHUNKS:
                pos_h[(c + 1) % 2] = start_pos(c + 1)
        in_h[slot].wait()
        if u + 1 < NU:
            nslot = (u + 1) % RING
            if out_h[nslot] is not None:
                out_h[nslot].wait()
            in_h[nslot] = start_in(u + 1, nslot)

        x_v, pos_v = x_bufs[slot], pos_bufs[c % 2]

        @plsc.parallel_loop(0, CHUNK * PPR, step=1, unroll=8)
        def _add(i):
            r = i // PPR
            col = (i % PPR) * L
            plsc.addupdate(x_v.at[r, pl.ds(col, L)], pos_v[r, pl.ds(col, L)])

        out_h[slot] = start_out(u, slot)

    for h in out_h:
        h.wait()


def kernel(x, pos_table):
    out = _sc_add(x.reshape(B * S, D), pos_table)
    return out.reshape(x.shape)
